# baseline (speedup 1.0000x reference)
import jax
import jax.numpy as jnp
from jax.experimental import pallas as pl


def _copy_body(hs_ref, pe_ref, m_ref, hs_out, pe_out, m_out):
    hs_out[...] = hs_ref[...]
    pe_out[...] = pe_ref[...]
    m_out[...] = m_ref[...]


def kernel(hidden_states, position_embeddings, attention_mask):
    b, q, h = hidden_states.shape
    hs2 = hidden_states.reshape(b, h)
    pe2 = position_embeddings.reshape(b, h)
    m2 = attention_mask.reshape(1, b)

    hs_o, pe_o, m_o = pl.pallas_call(
        _copy_body,
        out_shape=(
            jax.ShapeDtypeStruct(hs2.shape, hs2.dtype),
            jax.ShapeDtypeStruct(pe2.shape, pe2.dtype),
            jax.ShapeDtypeStruct(m2.shape, m2.dtype),
        ),
    )(hs2, pe2, m2)

    return (
        hs_o.reshape(hidden_states.shape),
        pe_o.reshape(position_embeddings.shape),
        m_o.reshape(attention_mask.shape),
    )
